# Initial kernel scaffold; baseline (speedup 1.0000x reference)
import jax
import jax.numpy as jnp
from jax.experimental import pallas as pl


def kernel(x, edge_index, W1, b1, Wmu, bmu, Wls, bls):
    raise NotImplementedError("write your pallas kernel here")


# speedup vs baseline: 7.0963x; 7.0963x over previous
import functools

import jax
import jax.numpy as jnp
from jax import lax
from jax.experimental import pallas as pl
from jax.experimental.pallas import tpu as pltpu
from jax.experimental.pallas import tpu_sc as plsc

N = 10000
NP = 10240
E = 320000
NC, NS = 2, 16
NW = NC * NS
EPW = E // NW
EPT = E // NS
CH = 80
RPT = NP // NS

_mesh = plsc.VectorSubcoreMesh(core_axis_name="c", subcore_axis_name="s",
                               num_cores=NC, num_subcores=NS)


@functools.partial(
    pl.kernel,
    out_type=jax.ShapeDtypeStruct((2, NP, 128), jnp.float32),
    mesh=_mesh,
    scratch_types=[
        pltpu.VMEM((CH,), jnp.int32),
        pltpu.VMEM((CH, 128), jnp.float32),
        pltpu.VMEM_SHARED((NP, 128), jnp.float32),
    ],
)
def _sc_degrees(src_hbm, dst_hbm, zeros_hbm, ones_hbm, out_hbm,
                idx_v, ones_v, acc):
    c = lax.axis_index("c")
    s = lax.axis_index("s")
    pltpu.sync_copy(ones_hbm, ones_v)
    pltpu.sync_copy(zeros_hbm.at[pl.ds(s * RPT, RPT)],
                    acc.at[pl.ds(s * RPT, RPT)])
    plsc.subcore_barrier()

    def run(idx_hbm):
        def body(i, _):
            pltpu.sync_copy(idx_hbm.at[pl.ds(s * EPT + i * CH, CH)], idx_v)
            pltpu.sync_copy(ones_v, acc.at[idx_v], add=True)
            return ()
        lax.fori_loop(0, EPT // CH, body, ())

    @pl.when(c == 0)
    def _():
        run(src_hbm)

    @pl.when(c == 1)
    def _():
        run(dst_hbm)

    plsc.subcore_barrier()
    pltpu.sync_copy(acc.at[pl.ds(s * RPT, RPT)],
                    out_hbm.at[c, pl.ds(s * RPT, RPT)])


@functools.partial(
    pl.kernel,
    out_type=jax.ShapeDtypeStruct((2, NP, 128), jnp.float32),
    mesh=_mesh,
    scratch_types=[
        pltpu.VMEM((CH,), jnp.int32),
        pltpu.VMEM((CH,), jnp.int32),
        pltpu.VMEM((CH, 128), jnp.float32),
        pltpu.VMEM_SHARED((NP, 128), jnp.float32),
        pltpu.SemaphoreType.DMA,
    ],
)
def _sc_aggregate(feat_hbm, src_hbm, dst_hbm, zeros_hbm, out_hbm,
                  idx_s, idx_d, rows, acc, sem):
    c = lax.axis_index("c")
    s = lax.axis_index("s")
    wid = s * NC + c
    pltpu.sync_copy(zeros_hbm.at[pl.ds(s * RPT, RPT)],
                    acc.at[pl.ds(s * RPT, RPT)])
    plsc.subcore_barrier()

    base = wid * EPW

    def body(i, _):
        off = base + i * CH
        pltpu.sync_copy(src_hbm.at[pl.ds(off, CH)], idx_s)
        pltpu.sync_copy(dst_hbm.at[pl.ds(off, CH)], idx_d)
        pltpu.async_copy(feat_hbm.at[idx_s], rows, sem).wait()
        pltpu.sync_copy(rows, acc.at[idx_d], add=True)
        return ()

    lax.fori_loop(0, EPW // CH, body, ())

    plsc.subcore_barrier()
    pltpu.sync_copy(acc.at[pl.ds(s * RPT, RPT)],
                    out_hbm.at[c, pl.ds(s * RPT, RPT)])


def _tc_norm_body(x_ref, cnt_ref, xs_ref):
    onn = lax.rsqrt(cnt_ref[0, 0:N, 0:1] + 1.0)
    xs_ref[...] = x_ref[...] * onn


def _tc_layer1_body(S_ref, xs_ref, cnt_ref, W_ref, b_ref, out_ref):
    S = S_ref[0, 0:N] + S_ref[1, 0:N] + xs_ref[...]
    inn = lax.rsqrt(cnt_ref[1, 0:N, 0:1] + 1.0)
    onn = lax.rsqrt(cnt_ref[0, 0:N, 0:1] + 1.0)
    h = jnp.dot(S, W_ref[...], preferred_element_type=jnp.float32)
    h = jnp.maximum(h * inn + b_ref[...], 0.0)
    out_ref[...] = h * onn


def _tc_layer2_body(S_ref, hs_ref, cnt_ref, W_ref, b_ref, out_ref):
    S = S_ref[0, 0:N] + S_ref[1, 0:N] + hs_ref[...]
    inn = lax.rsqrt(cnt_ref[1, 0:N, 0:1] + 1.0)
    o = jnp.dot(S, W_ref[...], preferred_element_type=jnp.float32)
    out_ref[...] = o * inn + b_ref[...]


def _tc_call(body, out_shape, *args):
    return pl.pallas_call(
        body, out_shape=jax.ShapeDtypeStruct(out_shape, jnp.float32)
    )(*args)


def kernel(x, edge_index, W1, b1, Wmu, bmu, Wls, bls):
    src = edge_index[0]
    dst = edge_index[1]
    zeros128 = jnp.zeros((NP, 128), jnp.float32)
    ones_rows = jnp.ones((CH, 128), jnp.float32)

    cnt = _sc_degrees(src, dst, zeros128, ones_rows)

    xs = _tc_call(_tc_norm_body, (N, 128), x, cnt)

    Sx = _sc_aggregate(xs, src, dst, zeros128)

    Wcat12 = jnp.concatenate([Wmu, Wls], axis=1)
    bcat12 = jnp.concatenate([bmu, bls]).reshape(1, 128)

    hs = _tc_call(_tc_layer1_body, (N, 128),
                  Sx, xs, cnt, W1, b1.reshape(1, 128))

    Sh = _sc_aggregate(hs, src, dst, zeros128)

    out = _tc_call(_tc_layer2_body, (N, 128),
                   Sh, hs, cnt, Wcat12, bcat12)

    return (out[:, :64], out[:, 64:])


# speedup vs baseline: 12.2949x; 1.7326x over previous
import functools

import jax
import jax.numpy as jnp
from jax import lax
from jax.experimental import pallas as pl
from jax.experimental.pallas import tpu as pltpu
from jax.experimental.pallas import tpu_sc as plsc

N = 10000
NP = 10240
E = 320000
NC, NS = 2, 16
NW = NC * NS
EPW = E // NW
EPT = E // NS
CH = 40
RPT = NP // NS

_mesh = plsc.VectorSubcoreMesh(core_axis_name="c", subcore_axis_name="s",
                               num_cores=NC, num_subcores=NS)


KBUF = 5


@functools.partial(
    pl.kernel,
    out_type=jax.ShapeDtypeStruct((2, NP, 128), jnp.float32),
    mesh=_mesh,
    scratch_types=[
        pltpu.VMEM((KBUF, CH), jnp.int32),
        pltpu.VMEM((CH, 16), jnp.float32),
        pltpu.VMEM((64, 16), jnp.float32),
        pltpu.VMEM((64, 128), jnp.float32),
        pltpu.VMEM_SHARED((NP, 16), jnp.float32),
    ] + [pltpu.SemaphoreType.DMA] * KBUF,
)
def _sc_degrees(src_hbm, dst_hbm, out_hbm,
                idx_v, ones_v, buf16, buf128, acc, *sems):
    c = lax.axis_index("c")
    s = lax.axis_index("s")
    nct = E // CH // NS

    one_row = jnp.where(lax.iota(jnp.int32, 16) == 0,
                        jnp.float32(1.0), jnp.float32(0.0))
    for j in range(CH):
        ones_v[j, :] = one_row

    zrow = jnp.zeros((16,), jnp.float32)

    def zbody(r, _):
        buf16[r, :] = zrow
        return ()

    lax.fori_loop(0, 64, zbody, ())

    def zstrip(t, _):
        pltpu.sync_copy(buf16, acc.at[pl.ds(s * RPT + t * 64, 64)])
        return ()

    lax.fori_loop(0, RPT // 64, zstrip, ())
    plsc.subcore_barrier()

    def run(idx_hbm):
        def body(i, _):
            cb = s * nct + i * KBUF
            handles = [
                pltpu.async_copy(idx_hbm.at[pl.ds((cb + k) * CH, CH)],
                                 idx_v.at[k], sems[k])
                for k in range(KBUF)
            ]
            for k in range(KBUF):
                handles[k].wait()
                pltpu.sync_copy(ones_v, acc.at[idx_v.at[k]], add=True)
            return ()
        lax.fori_loop(0, nct // KBUF, body, ())

    @pl.when(c == 0)
    def _():
        run(src_hbm)

    @pl.when(c == 1)
    def _():
        run(dst_hbm)

    plsc.subcore_barrier()

    def wstrip(t, _):
        pltpu.sync_copy(acc.at[pl.ds(s * RPT + t * 64, 64)], buf16)

        def wbody(r, _):
            buf128[r, 0:16] = buf16[r, :]
            return ()

        lax.fori_loop(0, 64, wbody, ())
        pltpu.sync_copy(buf128, out_hbm.at[c, pl.ds(s * RPT + t * 64, 64)])
        return ()

    lax.fori_loop(0, RPT // 64, wstrip, ())


NCHW = EPW // CH


@functools.partial(
    pl.kernel,
    out_type=jax.ShapeDtypeStruct((2, NP, 128), jnp.float32),
    mesh=_mesh,
    scratch_types=[
        pltpu.VMEM((KBUF, CH), jnp.int32),
        pltpu.VMEM((KBUF, CH), jnp.int32),
        pltpu.VMEM((KBUF, CH, 128), jnp.float32),
        pltpu.VMEM_SHARED((NP, 128), jnp.float32),
    ] + [pltpu.SemaphoreType.DMA] * (3 * KBUF),
)
def _sc_aggregate(feat_hbm, src_hbm, dst_hbm, zeros_hbm, out_hbm,
                  idx_s, idx_d, rows, acc, *sems):
    c = lax.axis_index("c")
    s = lax.axis_index("s")
    wid = s * NC + c
    pltpu.sync_copy(zeros_hbm.at[pl.ds(s * RPT, RPT)],
                    acc.at[pl.ds(s * RPT, RPT)])
    plsc.subcore_barrier()

    base = wid * EPW
    isem = sems[:KBUF]
    dsem = sems[KBUF:2 * KBUF]
    gsem = sems[2 * KBUF:]

    def body(i, _):
        cb = base + i * (KBUF * CH)
        ih = [pltpu.async_copy(src_hbm.at[pl.ds(cb + k * CH, CH)],
                               idx_s.at[k], isem[k]) for k in range(KBUF)]
        dh = [pltpu.async_copy(dst_hbm.at[pl.ds(cb + k * CH, CH)],
                               idx_d.at[k], dsem[k]) for k in range(KBUF)]
        gh = []
        for k in range(KBUF):
            ih[k].wait()
            gh.append(pltpu.async_copy(feat_hbm.at[idx_s.at[k]],
                                       rows.at[k], gsem[k]))
        for k in range(KBUF):
            gh[k].wait()
            dh[k].wait()
            pltpu.sync_copy(rows.at[k], acc.at[idx_d.at[k]], add=True)
        return ()

    lax.fori_loop(0, NCHW // KBUF, body, ())

    plsc.subcore_barrier()
    pltpu.sync_copy(acc.at[pl.ds(s * RPT, RPT)],
                    out_hbm.at[c, pl.ds(s * RPT, RPT)])


def _tc_norm_body(x_ref, cnt_ref, xs_ref):
    onn = lax.rsqrt(cnt_ref[0, 0:N, 0:1] + 1.0)
    xs_ref[...] = x_ref[...] * onn


def _tc_layer1_body(S_ref, xs_ref, cnt_ref, W_ref, b_ref, out_ref):
    S = S_ref[0, 0:N] + S_ref[1, 0:N] + xs_ref[...]
    inn = lax.rsqrt(cnt_ref[1, 0:N, 0:1] + 1.0)
    onn = lax.rsqrt(cnt_ref[0, 0:N, 0:1] + 1.0)
    h = jnp.dot(S, W_ref[...], preferred_element_type=jnp.float32)
    h = jnp.maximum(h * inn + b_ref[...], 0.0)
    out_ref[...] = h * onn


def _tc_layer2_body(S_ref, hs_ref, cnt_ref, W_ref, b_ref, out_ref):
    S = S_ref[0, 0:N] + S_ref[1, 0:N] + hs_ref[...]
    inn = lax.rsqrt(cnt_ref[1, 0:N, 0:1] + 1.0)
    o = jnp.dot(S, W_ref[...], preferred_element_type=jnp.float32)
    out_ref[...] = o * inn + b_ref[...]


def _tc_call(body, out_shape, *args):
    return pl.pallas_call(
        body, out_shape=jax.ShapeDtypeStruct(out_shape, jnp.float32)
    )(*args)


def kernel(x, edge_index, W1, b1, Wmu, bmu, Wls, bls):
    src = edge_index[0]
    dst = edge_index[1]
    zeros128 = jnp.zeros((NP, 128), jnp.float32)

    cnt = _sc_degrees(src, dst)

    xs = _tc_call(_tc_norm_body, (N, 128), x, cnt)

    Sx = _sc_aggregate(xs, src, dst, zeros128)

    Wcat12 = jnp.concatenate([Wmu, Wls], axis=1)
    bcat12 = jnp.concatenate([bmu, bls]).reshape(1, 128)

    hs = _tc_call(_tc_layer1_body, (N, 128),
                  Sx, xs, cnt, W1, b1.reshape(1, 128))

    Sh = _sc_aggregate(hs, src, dst, zeros128)

    out = _tc_call(_tc_layer2_body, (N, 128),
                   Sh, hs, cnt, Wcat12, bcat12)

    return (out[:, :64], out[:, 64:])


# speedup vs baseline: 14.9807x; 1.2184x over previous
import functools

import jax
import jax.numpy as jnp
from jax import lax
from jax.experimental import pallas as pl
from jax.experimental.pallas import tpu as pltpu
from jax.experimental.pallas import tpu_sc as plsc

N = 10000
NP = 10240
E = 320000
NC, NS = 2, 16
NW = NC * NS
EPW = E // NW
EPT = E // NS
CH = 40
RPT = NP // NS

_mesh = plsc.VectorSubcoreMesh(core_axis_name="c", subcore_axis_name="s",
                               num_cores=NC, num_subcores=NS)


KBUF = 5
DCH = 80


@functools.partial(
    pl.kernel,
    out_type=jax.ShapeDtypeStruct((2, NP, 128), jnp.float32),
    mesh=_mesh,
    scratch_types=[
        pltpu.VMEM((KBUF, DCH), jnp.int32),
        pltpu.VMEM((DCH, 16), jnp.float32),
        pltpu.VMEM((32, 16), jnp.float32),
        pltpu.VMEM((32, 128), jnp.float32),
        pltpu.VMEM_SHARED((NP, 16), jnp.float32),
    ] + [pltpu.SemaphoreType.DMA] * (2 * KBUF),
)
def _sc_degrees(src_hbm, dst_hbm, out_hbm,
                idx_v, ones_v, buf16, buf128, acc, *sems):
    c = lax.axis_index("c")
    s = lax.axis_index("s")
    nct = E // DCH // NS
    isem = sems[:KBUF]
    ssem = sems[KBUF:]

    one_row = jnp.where(lax.iota(jnp.int32, 16) == 0,
                        jnp.float32(1.0), jnp.float32(0.0))
    for j in range(DCH):
        ones_v[j, :] = one_row

    zrow = jnp.zeros((16,), jnp.float32)

    def zbody(r, _):
        buf16[r, :] = zrow
        return ()

    lax.fori_loop(0, 32, zbody, ())

    def zstrip(t, _):
        pltpu.sync_copy(buf16, acc.at[pl.ds(s * RPT + t * 32, 32)])
        return ()

    lax.fori_loop(0, RPT // 32, zstrip, ())
    plsc.subcore_barrier()

    def run(idx_hbm):
        def body(i, _):
            cb = s * nct + i * KBUF
            ih = [
                pltpu.async_copy(idx_hbm.at[pl.ds((cb + k) * DCH, DCH)],
                                 idx_v.at[k], isem[k])
                for k in range(KBUF)
            ]
            sh = []
            for k in range(KBUF):
                ih[k].wait()
                sh.append(pltpu.async_copy(ones_v, acc.at[idx_v.at[k]],
                                           ssem[k], add=True))
            for k in range(KBUF):
                sh[k].wait()
            return ()
        lax.fori_loop(0, nct // KBUF, body, ())

    @pl.when(c == 0)
    def _():
        run(src_hbm)

    @pl.when(c == 1)
    def _():
        run(dst_hbm)

    plsc.subcore_barrier()

    def wstrip(t, _):
        pltpu.sync_copy(acc.at[pl.ds(s * RPT + t * 32, 32)], buf16)

        def wbody(r, _):
            buf128[r, 0:16] = buf16[r, :]
            return ()

        lax.fori_loop(0, 32, wbody, ())
        pltpu.sync_copy(buf128, out_hbm.at[c, pl.ds(s * RPT + t * 32, 32)])
        return ()

    lax.fori_loop(0, RPT // 32, wstrip, ())


NCHW = EPW // CH


G2 = 2 * KBUF


@functools.partial(
    pl.kernel,
    out_type=jax.ShapeDtypeStruct((2, NP, 128), jnp.float32),
    mesh=_mesh,
    scratch_types=[
        pltpu.VMEM((G2, CH), jnp.int32),
        pltpu.VMEM((G2, CH), jnp.int32),
        pltpu.VMEM((KBUF, CH, 128), jnp.float32),
        pltpu.VMEM_SHARED((NP, 128), jnp.float32),
    ] + [pltpu.SemaphoreType.DMA] * (2 * G2 + 2 * KBUF),
)
def _sc_aggregate(feat_hbm, src_hbm, dst_hbm, zeros_hbm, out_hbm,
                  idx_s, idx_d, rows, acc, *sems):
    c = lax.axis_index("c")
    s = lax.axis_index("s")
    wid = s * NC + c
    pltpu.sync_copy(zeros_hbm.at[pl.ds(s * RPT, RPT)],
                    acc.at[pl.ds(s * RPT, RPT)])
    plsc.subcore_barrier()

    base = wid * EPW
    isem = sems[:G2]
    dsem = sems[G2:2 * G2]
    gsem = sems[2 * G2:2 * G2 + KBUF]
    ssem = sems[2 * G2 + KBUF:]

    def body(i, _):
        cb = base + i * (G2 * CH)
        ih = [pltpu.async_copy(src_hbm.at[pl.ds(cb + j * CH, CH)],
                               idx_s.at[j], isem[j]) for j in range(G2)]
        dh = [pltpu.async_copy(dst_hbm.at[pl.ds(cb + j * CH, CH)],
                               idx_d.at[j], dsem[j]) for j in range(G2)]
        gh = []
        for k in range(KBUF):
            ih[k].wait()
            gh.append(pltpu.async_copy(feat_hbm.at[idx_s.at[k]],
                                       rows.at[k], gsem[k]))
        sh = []
        for k in range(KBUF):
            gh[k].wait()
            dh[k].wait()
            sh.append(pltpu.async_copy(rows.at[k], acc.at[idx_d.at[k]],
                                       ssem[k], add=True))
        ghB = []
        for k in range(KBUF):
            j = KBUF + k
            ih[j].wait()
            sh[k].wait()
            ghB.append(pltpu.async_copy(feat_hbm.at[idx_s.at[j]],
                                        rows.at[k], gsem[k]))
        shB = []
        for k in range(KBUF):
            j = KBUF + k
            ghB[k].wait()
            dh[j].wait()
            shB.append(pltpu.async_copy(rows.at[k], acc.at[idx_d.at[j]],
                                        ssem[k], add=True))
        for k in range(KBUF):
            shB[k].wait()
        return ()

    lax.fori_loop(0, EPW // CH // G2, body, ())

    plsc.subcore_barrier()
    pltpu.sync_copy(acc.at[pl.ds(s * RPT, RPT)],
                    out_hbm.at[c, pl.ds(s * RPT, RPT)])


def _tc_norm_body(x_ref, cnt_ref, xs_ref):
    onn = lax.rsqrt(cnt_ref[0, 0:N, 0:1] + 1.0)
    xs_ref[...] = x_ref[...] * onn


def _tc_layer1_body(S_ref, xs_ref, cnt_ref, W_ref, b_ref, out_ref):
    S = S_ref[0, 0:N] + S_ref[1, 0:N] + xs_ref[...]
    inn = lax.rsqrt(cnt_ref[1, 0:N, 0:1] + 1.0)
    onn = lax.rsqrt(cnt_ref[0, 0:N, 0:1] + 1.0)
    h = jnp.dot(S, W_ref[...], preferred_element_type=jnp.float32)
    h = jnp.maximum(h * inn + b_ref[...], 0.0)
    out_ref[...] = h * onn


def _tc_layer2_body(S_ref, hs_ref, cnt_ref, W_ref, b_ref, out_ref):
    S = S_ref[0, 0:N] + S_ref[1, 0:N] + hs_ref[...]
    inn = lax.rsqrt(cnt_ref[1, 0:N, 0:1] + 1.0)
    o = jnp.dot(S, W_ref[...], preferred_element_type=jnp.float32)
    out_ref[...] = o * inn + b_ref[...]


def _tc_call(body, out_shape, *args):
    return pl.pallas_call(
        body, out_shape=jax.ShapeDtypeStruct(out_shape, jnp.float32)
    )(*args)


def kernel(x, edge_index, W1, b1, Wmu, bmu, Wls, bls):
    src = edge_index[0]
    dst = edge_index[1]
    zeros128 = jnp.zeros((NP, 128), jnp.float32)

    cnt = _sc_degrees(src, dst)

    xs = _tc_call(_tc_norm_body, (N, 128), x, cnt)

    Sx = _sc_aggregate(xs, src, dst, zeros128)

    Wcat12 = jnp.concatenate([Wmu, Wls], axis=1)
    bcat12 = jnp.concatenate([bmu, bls]).reshape(1, 128)

    hs = _tc_call(_tc_layer1_body, (N, 128),
                  Sx, xs, cnt, W1, b1.reshape(1, 128))

    Sh = _sc_aggregate(hs, src, dst, zeros128)

    out = _tc_call(_tc_layer2_body, (N, 128),
                   Sh, hs, cnt, Wcat12, bcat12)

    return (out[:, :64], out[:, 64:])


# speedup vs baseline: 15.3016x; 1.0214x over previous
import functools

import jax
import jax.numpy as jnp
from jax import lax
from jax.experimental import pallas as pl
from jax.experimental.pallas import tpu as pltpu
from jax.experimental.pallas import tpu_sc as plsc

N = 10000
NP = 10240
E = 320000
NC, NS = 2, 16
NW = NC * NS
EPW = E // NW
EPT = E // NS
CH = 40
RPT = NP // NS

_mesh = plsc.VectorSubcoreMesh(core_axis_name="c", subcore_axis_name="s",
                               num_cores=NC, num_subcores=NS)


KBUF = 5
DCH = 80


@functools.partial(
    pl.kernel,
    out_type=jax.ShapeDtypeStruct((2, NP, 128), jnp.float32),
    mesh=_mesh,
    scratch_types=[
        pltpu.VMEM((KBUF, DCH), jnp.int32),
        pltpu.VMEM((DCH, 16), jnp.float32),
        pltpu.VMEM((32, 16), jnp.float32),
        pltpu.VMEM((32, 128), jnp.float32),
        pltpu.VMEM_SHARED((NP, 16), jnp.float32),
    ] + [pltpu.SemaphoreType.DMA] * (2 * KBUF),
)
def _sc_degrees(src_hbm, dst_hbm, out_hbm,
                idx_v, ones_v, buf16, buf128, acc, *sems):
    c = lax.axis_index("c")
    s = lax.axis_index("s")
    nct = E // DCH // NS
    isem = sems[:KBUF]
    ssem = sems[KBUF:]

    one_row = jnp.where(lax.iota(jnp.int32, 16) == 0,
                        jnp.float32(1.0), jnp.float32(0.0))
    for j in range(DCH):
        ones_v[j, :] = one_row

    zrow = jnp.zeros((16,), jnp.float32)

    def zbody(r, _):
        buf16[r, :] = zrow
        return ()

    lax.fori_loop(0, 32, zbody, ())

    def zstrip(t, _):
        pltpu.sync_copy(buf16, acc.at[pl.ds(s * RPT + t * 32, 32)])
        return ()

    lax.fori_loop(0, RPT // 32, zstrip, ())
    plsc.subcore_barrier()

    def run(idx_hbm):
        def body(i, _):
            cb = s * nct + i * KBUF
            ih = [
                pltpu.async_copy(idx_hbm.at[pl.ds((cb + k) * DCH, DCH)],
                                 idx_v.at[k], isem[k])
                for k in range(KBUF)
            ]
            sh = []
            for k in range(KBUF):
                ih[k].wait()
                sh.append(pltpu.async_copy(ones_v, acc.at[idx_v.at[k]],
                                           ssem[k], add=True))
            for k in range(KBUF):
                sh[k].wait()
            return ()
        lax.fori_loop(0, nct // KBUF, body, ())

    @pl.when(c == 0)
    def _():
        run(src_hbm)

    @pl.when(c == 1)
    def _():
        run(dst_hbm)

    plsc.subcore_barrier()

    def wstrip(t, _):
        pltpu.sync_copy(acc.at[pl.ds(s * RPT + t * 32, 32)], buf16)

        def wbody(r, _):
            buf128[r, 0:16] = buf16[r, :]
            return ()

        lax.fori_loop(0, 32, wbody, ())
        pltpu.sync_copy(buf128, out_hbm.at[c, pl.ds(s * RPT + t * 32, 32)])
        return ()

    lax.fori_loop(0, RPT // 32, wstrip, ())


NCHW = EPW // CH


G2 = 2 * KBUF


@functools.partial(
    pl.kernel,
    out_type=jax.ShapeDtypeStruct((2, NP, 128), jnp.float32),
    mesh=_mesh,
    scratch_types=[
        pltpu.VMEM((G2, CH), jnp.int32),
        pltpu.VMEM((G2, CH), jnp.int32),
        pltpu.VMEM((KBUF, CH, 128), jnp.float32),
        pltpu.VMEM_SHARED((NP, 128), jnp.float32),
    ] + [pltpu.SemaphoreType.DMA] * (2 * G2 + 2 * KBUF),
)
def _sc_aggregate(feat_hbm, src_hbm, dst_hbm, zeros_hbm, out_hbm,
                  idx_s, idx_d, rows, acc, *sems):
    c = lax.axis_index("c")
    s = lax.axis_index("s")
    wid = s * NC + c
    pltpu.sync_copy(zeros_hbm.at[pl.ds(s * RPT, RPT)],
                    acc.at[pl.ds(s * RPT, RPT)])

    base = wid * EPW
    isem = sems[:G2]
    dsem = sems[G2:2 * G2]
    gsem = sems[2 * G2:2 * G2 + KBUF]
    ssem = sems[2 * G2 + KBUF:]

    zrow_i = jnp.zeros((16,), jnp.int32)
    for j in range(G2):
        idx_d[j, 0:16] = zrow_i
        idx_d[j, 16:32] = zrow_i
        idx_d[j, 24:40] = zrow_i
    for k in range(KBUF):
        pltpu.sync_copy(zeros_hbm.at[pl.ds(0, CH)], rows.at[k])
    plsc.subcore_barrier()
    for k in range(KBUF):
        pltpu.async_copy(rows.at[k], acc.at[idx_d.at[KBUF + k]],
                         ssem[k], add=True)

    def _drain(k):
        pltpu.make_async_copy(rows.at[k],
                              acc.at[idx_d.at[KBUF + k]], ssem[k]).wait()

    def body(i, _):
        cb = base + i * (G2 * CH)
        ih = [pltpu.async_copy(src_hbm.at[pl.ds(cb + j * CH, CH)],
                               idx_s.at[j], isem[j]) for j in range(G2)]
        dh = [pltpu.async_copy(dst_hbm.at[pl.ds(cb + j * CH, CH)],
                               idx_d.at[j], dsem[j]) for j in range(G2)]
        gh = []
        for k in range(KBUF):
            ih[k].wait()
            _drain(k)
            gh.append(pltpu.async_copy(feat_hbm.at[idx_s.at[k]],
                                       rows.at[k], gsem[k]))
        sh = []
        for k in range(KBUF):
            gh[k].wait()
            dh[k].wait()
            sh.append(pltpu.async_copy(rows.at[k], acc.at[idx_d.at[k]],
                                       ssem[k], add=True))
        ghB = []
        for k in range(KBUF):
            j = KBUF + k
            ih[j].wait()
            sh[k].wait()
            ghB.append(pltpu.async_copy(feat_hbm.at[idx_s.at[j]],
                                        rows.at[k], gsem[k]))
        for k in range(KBUF):
            j = KBUF + k
            ghB[k].wait()
            dh[j].wait()
            pltpu.async_copy(rows.at[k], acc.at[idx_d.at[j]],
                             ssem[k], add=True)
        return ()

    lax.fori_loop(0, EPW // CH // G2, body, ())

    for k in range(KBUF):
        _drain(k)
    plsc.subcore_barrier()
    pltpu.sync_copy(acc.at[pl.ds(s * RPT, RPT)],
                    out_hbm.at[c, pl.ds(s * RPT, RPT)])


def _tc_norm_body(x_ref, cnt_ref, xs_ref):
    onn = lax.rsqrt(cnt_ref[0, 0:N, 0:1] + 1.0)
    xs_ref[...] = x_ref[...] * onn


def _tc_layer1_body(S_ref, xs_ref, cnt_ref, W_ref, b_ref, out_ref):
    S = S_ref[0, 0:N] + S_ref[1, 0:N] + xs_ref[...]
    inn = lax.rsqrt(cnt_ref[1, 0:N, 0:1] + 1.0)
    onn = lax.rsqrt(cnt_ref[0, 0:N, 0:1] + 1.0)
    h = jnp.dot(S, W_ref[...], preferred_element_type=jnp.float32)
    h = jnp.maximum(h * inn + b_ref[...], 0.0)
    out_ref[...] = h * onn


def _tc_layer2_body(S_ref, hs_ref, cnt_ref, W_ref, b_ref, out_ref):
    S = S_ref[0, 0:N] + S_ref[1, 0:N] + hs_ref[...]
    inn = lax.rsqrt(cnt_ref[1, 0:N, 0:1] + 1.0)
    o = jnp.dot(S, W_ref[...], preferred_element_type=jnp.float32)
    out_ref[...] = o * inn + b_ref[...]


def _tc_call(body, out_shape, *args):
    return pl.pallas_call(
        body, out_shape=jax.ShapeDtypeStruct(out_shape, jnp.float32)
    )(*args)


def kernel(x, edge_index, W1, b1, Wmu, bmu, Wls, bls):
    src = edge_index[0]
    dst = edge_index[1]
    zeros128 = jnp.zeros((NP, 128), jnp.float32)

    cnt = _sc_degrees(src, dst)

    xs = _tc_call(_tc_norm_body, (N, 128), x, cnt)

    Sx = _sc_aggregate(xs, src, dst, zeros128)

    Wcat12 = jnp.concatenate([Wmu, Wls], axis=1)
    bcat12 = jnp.concatenate([bmu, bls]).reshape(1, 128)

    hs = _tc_call(_tc_layer1_body, (N, 128),
                  Sx, xs, cnt, W1, b1.reshape(1, 128))

    Sh = _sc_aggregate(hs, src, dst, zeros128)

    out = _tc_call(_tc_layer2_body, (N, 128),
                   Sh, hs, cnt, Wcat12, bcat12)

    return (out[:, :64], out[:, 64:])


# speedup vs baseline: 15.9815x; 1.0444x over previous
import functools

import jax
import jax.numpy as jnp
from jax import lax
from jax.experimental import pallas as pl
from jax.experimental.pallas import tpu as pltpu
from jax.experimental.pallas import tpu_sc as plsc

N = 10000
NP = 10240
E = 320000
NC, NS = 2, 16
NW = NC * NS
EPW = E // NW
EPT = E // NS
CH = 40
RPT = NP // NS

_mesh = plsc.VectorSubcoreMesh(core_axis_name="c", subcore_axis_name="s",
                               num_cores=NC, num_subcores=NS)


KBUF = 5
DCH = 80


@functools.partial(
    pl.kernel,
    out_type=jax.ShapeDtypeStruct((2, NP, 128), jnp.float32),
    mesh=_mesh,
    scratch_types=[
        pltpu.VMEM((KBUF, DCH), jnp.int32),
        pltpu.VMEM((DCH, 16), jnp.float32),
        pltpu.VMEM((DCH, 16), jnp.float32),
        pltpu.VMEM((32, 16), jnp.float32),
        pltpu.VMEM((32, 128), jnp.float32),
        pltpu.VMEM_SHARED((NP, 16), jnp.float32),
    ] + [pltpu.SemaphoreType.DMA] * (2 * KBUF),
)
def _sc_degrees(src_hbm, dst_hbm, out_hbm,
                idx_v, ones_v, zsrc, buf16, buf128, acc, *sems):
    c = lax.axis_index("c")
    s = lax.axis_index("s")
    nct = E // DCH // NS
    isem = sems[:KBUF]
    ssem = sems[KBUF:]

    one_row = jnp.where(lax.iota(jnp.int32, 16) == 0,
                        jnp.float32(1.0), jnp.float32(0.0))
    for j in range(DCH):
        ones_v[j, :] = one_row

    zrow = jnp.zeros((16,), jnp.float32)

    def zbody(r, _):
        buf16[r, :] = zrow
        return ()

    lax.fori_loop(0, 32, zbody, ())

    def zstrip(t, _):
        pltpu.sync_copy(buf16, acc.at[pl.ds(s * RPT + t * 32, 32)])
        return ()

    lax.fori_loop(0, RPT // 32, zstrip, ())
    plsc.subcore_barrier()

    zrow_i = jnp.zeros((16,), jnp.int32)
    for j in range(KBUF):
        for b in range(0, DCH, 16):
            idx_v[j, b:b + 16] = zrow_i

    def zsbody(r, _):
        zsrc[r, :] = zrow
        return ()

    lax.fori_loop(0, DCH, zsbody, ())
    plsc.subcore_barrier()
    for k in range(KBUF):
        pltpu.async_copy(zsrc, acc.at[idx_v.at[k]], ssem[k], add=True)

    def _drain(k):
        pltpu.make_async_copy(zsrc, acc.at[idx_v.at[k]], ssem[k]).wait()

    def run(idx_hbm):
        def body(i, _):
            cb = s * nct + i * KBUF
            ih = []
            for k in range(KBUF):
                _drain(k)
                ih.append(pltpu.async_copy(
                    idx_hbm.at[pl.ds((cb + k) * DCH, DCH)],
                    idx_v.at[k], isem[k]))
            for k in range(KBUF):
                ih[k].wait()
                pltpu.async_copy(ones_v, acc.at[idx_v.at[k]],
                                 ssem[k], add=True)
            return ()
        lax.fori_loop(0, nct // KBUF, body, ())

    @pl.when(c == 0)
    def _():
        run(src_hbm)

    @pl.when(c == 1)
    def _():
        run(dst_hbm)

    for k in range(KBUF):
        _drain(k)
    plsc.subcore_barrier()

    def wstrip(t, _):
        pltpu.sync_copy(acc.at[pl.ds(s * RPT + t * 32, 32)], buf16)

        def wbody(r, _):
            buf128[r, 0:16] = buf16[r, :]
            return ()

        lax.fori_loop(0, 32, wbody, ())
        pltpu.sync_copy(buf128, out_hbm.at[c, pl.ds(s * RPT + t * 32, 32)])
        return ()

    lax.fori_loop(0, RPT // 32, wstrip, ())


NCHW = EPW // CH


G2 = 2 * KBUF


@functools.partial(
    pl.kernel,
    out_type=jax.ShapeDtypeStruct((2, NP, 128), jnp.float32),
    mesh=_mesh,
    scratch_types=[
        pltpu.VMEM((G2, CH), jnp.int32),
        pltpu.VMEM((G2, CH), jnp.int32),
        pltpu.VMEM((KBUF, CH, 128), jnp.float32),
        pltpu.VMEM_SHARED((NP, 128), jnp.float32),
    ] + [pltpu.SemaphoreType.DMA] * (2 * G2 + 2 * KBUF),
)
def _sc_aggregate(feat_hbm, src_hbm, dst_hbm, zeros_hbm, out_hbm,
                  idx_s, idx_d, rows, acc, *sems):
    c = lax.axis_index("c")
    s = lax.axis_index("s")
    wid = s * NC + c
    pltpu.sync_copy(zeros_hbm.at[pl.ds(s * RPT, RPT)],
                    acc.at[pl.ds(s * RPT, RPT)])

    base = wid * EPW
    isem = sems[:G2]
    dsem = sems[G2:2 * G2]
    gsem = sems[2 * G2:2 * G2 + KBUF]
    ssem = sems[2 * G2 + KBUF:]

    zrow_i = jnp.zeros((16,), jnp.int32)
    for j in range(G2):
        idx_d[j, 0:16] = zrow_i
        idx_d[j, 16:32] = zrow_i
        idx_d[j, 24:40] = zrow_i
    for k in range(KBUF):
        pltpu.sync_copy(zeros_hbm.at[pl.ds(0, CH)], rows.at[k])
    plsc.subcore_barrier()
    for k in range(KBUF):
        pltpu.async_copy(rows.at[k], acc.at[idx_d.at[KBUF + k]],
                         ssem[k], add=True)

    def _drain(k):
        pltpu.make_async_copy(rows.at[k],
                              acc.at[idx_d.at[KBUF + k]], ssem[k]).wait()

    def body(i, _):
        cb = base + i * (G2 * CH)
        ih = [pltpu.async_copy(src_hbm.at[pl.ds(cb + j * CH, CH)],
                               idx_s.at[j], isem[j]) for j in range(G2)]
        dh = [pltpu.async_copy(dst_hbm.at[pl.ds(cb + j * CH, CH)],
                               idx_d.at[j], dsem[j]) for j in range(KBUF)]
        gh = []
        dhB = []
        for k in range(KBUF):
            ih[k].wait()
            _drain(k)
            j = KBUF + k
            dhB.append(pltpu.async_copy(dst_hbm.at[pl.ds(cb + j * CH, CH)],
                                        idx_d.at[j], dsem[j]))
            gh.append(pltpu.async_copy(feat_hbm.at[idx_s.at[k]],
                                       rows.at[k], gsem[k]))
        sh = []
        for k in range(KBUF):
            gh[k].wait()
            dh[k].wait()
            sh.append(pltpu.async_copy(rows.at[k], acc.at[idx_d.at[k]],
                                       ssem[k], add=True))
        ghB = []
        for k in range(KBUF):
            j = KBUF + k
            ih[j].wait()
            sh[k].wait()
            ghB.append(pltpu.async_copy(feat_hbm.at[idx_s.at[j]],
                                        rows.at[k], gsem[k]))
        for k in range(KBUF):
            j = KBUF + k
            ghB[k].wait()
            dhB[k].wait()
            pltpu.async_copy(rows.at[k], acc.at[idx_d.at[j]],
                             ssem[k], add=True)
        return ()

    lax.fori_loop(0, EPW // CH // G2, body, ())

    for k in range(KBUF):
        _drain(k)
    plsc.subcore_barrier()
    pltpu.sync_copy(acc.at[pl.ds(s * RPT, RPT)],
                    out_hbm.at[c, pl.ds(s * RPT, RPT)])


def _tc_norm_body(x_ref, cnt_ref, xs_ref):
    onn = lax.rsqrt(cnt_ref[0, 0:N, 0:1] + 1.0)
    xs_ref[...] = x_ref[...] * onn


def _tc_layer2_body2(S_ref, hs_ref, cnt_ref, W_ref, b_ref, mu_ref, ls_ref):
    S = S_ref[0, 0:N] + S_ref[1, 0:N] + hs_ref[...]
    inn = lax.rsqrt(cnt_ref[1, 0:N, 0:1] + 1.0)
    o = jnp.dot(S, W_ref[...], preferred_element_type=jnp.float32)
    o = o * inn + b_ref[...]
    mu_ref[...] = o[:, 0:64]
    ls_ref[...] = o[:, 64:128]


def _tc_layer1_body(S_ref, xs_ref, cnt_ref, W_ref, b_ref, out_ref):
    S = S_ref[0, 0:N] + S_ref[1, 0:N] + xs_ref[...]
    inn = lax.rsqrt(cnt_ref[1, 0:N, 0:1] + 1.0)
    onn = lax.rsqrt(cnt_ref[0, 0:N, 0:1] + 1.0)
    h = jnp.dot(S, W_ref[...], preferred_element_type=jnp.float32)
    h = jnp.maximum(h * inn + b_ref[...], 0.0)
    out_ref[...] = h * onn


def _tc_layer2_body(S_ref, hs_ref, cnt_ref, W_ref, b_ref, out_ref):
    S = S_ref[0, 0:N] + S_ref[1, 0:N] + hs_ref[...]
    inn = lax.rsqrt(cnt_ref[1, 0:N, 0:1] + 1.0)
    o = jnp.dot(S, W_ref[...], preferred_element_type=jnp.float32)
    out_ref[...] = o * inn + b_ref[...]


def _tc_call(body, out_shape, *args):
    return pl.pallas_call(
        body, out_shape=jax.ShapeDtypeStruct(out_shape, jnp.float32)
    )(*args)


def kernel(x, edge_index, W1, b1, Wmu, bmu, Wls, bls):
    src = edge_index[0]
    dst = edge_index[1]
    zeros128 = jnp.zeros((NP, 128), jnp.float32)

    cnt = _sc_degrees(src, dst)[:, :, 0:8]

    xs = _tc_call(_tc_norm_body, (N, 128), x, cnt)

    Sx = _sc_aggregate(xs, src, dst, zeros128)

    Wcat12 = jnp.concatenate([Wmu, Wls], axis=1)
    bcat12 = jnp.concatenate([bmu, bls]).reshape(1, 128)

    hs = _tc_call(_tc_layer1_body, (N, 128),
                  Sx, xs, cnt, W1, b1.reshape(1, 128))

    Sh = _sc_aggregate(hs, src, dst, zeros128)

    mu, ls = pl.pallas_call(
        _tc_layer2_body2,
        out_shape=[jax.ShapeDtypeStruct((N, 64), jnp.float32),
                   jax.ShapeDtypeStruct((N, 64), jnp.float32)],
    )(Sh, hs, cnt, Wcat12, bcat12)

    return (mu, ls)
